# Initial kernel scaffold; baseline (speedup 1.0000x reference)
import functools

import jax
import jax.numpy as jnp
from jax import lax
from jax.experimental import pallas as pl
from jax.experimental.pallas import tpu as pltpu
from jax.experimental.pallas import tpu_sc as plsc

N = 10000
E = 320000
R = 8
NB_BASES = 30

NSC = 2
NT = 16
L = 16

CH = 1680
NCHUNK = 6
CPS = NCHUNK // NSC
NPAD = NCHUNK * CH
CHR = CH * R
TRASH = CHR
ROWS_PT = CHR // NT

EPT = E // NT
SEG = 4000
NSEG = EPT // SEG
KB = 128
NBLK_CAP = (EPT + KB - 1) // KB


def _sc_agg(feat, src, dst, et, zfeat, zcnt, ones_h):
    mesh = plsc.VectorSubcoreMesh(
        core_axis_name="c", subcore_axis_name="s",
        num_cores=NSC, num_subcores=NT)

    @functools.partial(
        pl.kernel,
        out_type=[
            jax.ShapeDtypeStruct((R * NPAD, 128), jnp.float32),
            jax.ShapeDtypeStruct((R * NPAD, 16), jnp.float32),
        ],
        mesh=mesh,
        scratch_types=[
            pltpu.VMEM((SEG,), jnp.int32),
            pltpu.VMEM((SEG,), jnp.int32),
            pltpu.VMEM((SEG,), jnp.int32),
            pltpu.VMEM((NBLK_CAP, KB), jnp.int32),
            pltpu.VMEM((NBLK_CAP, KB), jnp.int32),
            pltpu.VMEM((KB, 128), jnp.float32),
            pltpu.VMEM((KB, 16), jnp.float32),
            pltpu.VMEM_SHARED((CHR + 8, 128), jnp.float32),
            pltpu.VMEM_SHARED((CHR + 8, 16), jnp.float32),
            pltpu.SemaphoreType.DMA,
        ],
    )
    def k(feat_h, src_h, dst_h, et_h, zf_h, zc_h, ones_hbm, a_out, c_out,
          sbuf, dbuf, tbuf, gidx, ridx, rows, ones_v, acc_sh, cnt_sh, sem):
        cid = lax.axis_index("c")
        tid = lax.axis_index("s")
        pltpu.sync_copy(ones_hbm, ones_v)
        iota = lax.iota(jnp.int32, L)
        r0 = tid * ROWS_PT

        for ci in range(CPS):
            chunk = cid * CPS + ci
            lo = chunk * CH

            pltpu.sync_copy(zf_h.at[pl.ds(r0, ROWS_PT)],
                            acc_sh.at[pl.ds(r0, ROWS_PT)])
            pltpu.sync_copy(zc_h.at[pl.ds(r0, ROWS_PT)],
                            cnt_sh.at[pl.ds(r0, ROWS_PT)])

            @pl.when(tid == 0)
            def _():
                pltpu.sync_copy(zf_h.at[pl.ds(TRASH, 8)],
                                acc_sh.at[pl.ds(TRASH, 8)])
                pltpu.sync_copy(zc_h.at[pl.ds(TRASH, 8)],
                                cnt_sh.at[pl.ds(TRASH, 8)])

            plsc.subcore_barrier()

            off = jnp.int32(0)
            for sg in range(NSEG):
                sb = tid * EPT + sg * SEG
                pltpu.sync_copy(src_h.at[pl.ds(sb, SEG)], sbuf)
                pltpu.sync_copy(dst_h.at[pl.ds(sb, SEG)], dbuf)
                pltpu.sync_copy(et_h.at[pl.ds(sb, SEG)], tbuf)

                def grp(g, off):
                    s = sbuf[pl.ds(g * L, L)]
                    d = dbuf[pl.ds(g * L, L)]
                    t = tbuf[pl.ds(g * L, L)]
                    m = (d >= lo) & (d < lo + CH)
                    rl = t * CH + (d - lo)
                    mi = m.astype(jnp.int32)
                    idx = off + plsc.cumsum(mi) - 1
                    plsc.store_scatter(gidx, [idx >> 7, idx & 127], s, mask=m)
                    plsc.store_scatter(ridx, [idx >> 7, idx & 127], rl, mask=m)
                    return off + jnp.sum(mi)

                off = lax.fori_loop(0, SEG // L, grp, off)

            nblk = (off + KB - 1) // KB
            pend = nblk * KB
            for kk in range(KB // L):
                pidx = off + kk * L + iota
                pm = pidx < pend
                plsc.store_scatter(gidx, [pidx >> 7, pidx & 127],
                                   jnp.zeros((L,), jnp.int32), mask=pm)
                plsc.store_scatter(ridx, [pidx >> 7, pidx & 127],
                                   jnp.full((L,), TRASH, jnp.int32), mask=pm)

            def blk(b, carry):
                gi = gidx.at[b]
                ri = ridx.at[b]
                pltpu.async_copy(feat_h.at[gi], rows, sem).wait()
                pltpu.sync_copy(rows, acc_sh.at[ri], add=True)
                pltpu.sync_copy(ones_v, cnt_sh.at[ri], add=True)
                return carry

            lax.fori_loop(0, nblk, blk, jnp.int32(0))
            plsc.subcore_barrier()

            rel = tid // 2
            d0 = rel * NPAD + chunk * CH + (tid % 2) * ROWS_PT
            pltpu.sync_copy(acc_sh.at[pl.ds(r0, ROWS_PT)],
                            a_out.at[pl.ds(d0, ROWS_PT)])
            pltpu.sync_copy(cnt_sh.at[pl.ds(r0, ROWS_PT)],
                            c_out.at[pl.ds(d0, ROWS_PT)])

    return k(feat, src, dst, et, zfeat, zcnt, ones_h)


def _wcomb(comp_p, basis_flat):
    cols = basis_flat.shape[1]

    def body(c_ref, b_ref, o_ref):
        o_ref[...] = jnp.dot(c_ref[...], b_ref[...],
                             preferred_element_type=jnp.float32)

    return pl.pallas_call(
        body,
        out_shape=jax.ShapeDtypeStruct((R, cols), jnp.float32),
    )(comp_p, basis_flat)


BLK = 720


def _conv_matmul(a3, c3, xfeat, w3, root, bias, out_dim, leaky):
    nb = NPAD // BLK

    def body(a_ref, c_ref, x_ref, w_ref, rt_ref, b_ref, o_ref):
        r = pl.program_id(1)

        @pl.when(r == 0)
        def _():
            o_ref[...] = (
                jnp.dot(x_ref[...], rt_ref[...],
                        preferred_element_type=jnp.float32) + b_ref[...])

        a = a_ref[0]
        c = c_ref[0][:, :1]
        inv = 1.0 / jnp.maximum(c, 1.0)
        o_ref[...] += jnp.dot(a * inv, w_ref[0],
                              preferred_element_type=jnp.float32)

        if leaky:
            @pl.when(r == R - 1)
            def _():
                v = o_ref[...]
                o_ref[...] = jnp.where(v > 0, v, 0.01 * v)

    return pl.pallas_call(
        body,
        grid=(nb, R),
        in_specs=[
            pl.BlockSpec((1, BLK, 128), lambda i, r: (r, i, 0)),
            pl.BlockSpec((1, BLK, 16), lambda i, r: (r, i, 0)),
            pl.BlockSpec((BLK, 128), lambda i, r: (i, 0)),
            pl.BlockSpec((1, 128, out_dim), lambda i, r: (r, 0, 0)),
            pl.BlockSpec((128, out_dim), lambda i, r: (0, 0)),
            pl.BlockSpec((1, out_dim), lambda i, r: (0, 0)),
        ],
        out_specs=pl.BlockSpec((BLK, out_dim), lambda i, r: (i, 0)),
        out_shape=jax.ShapeDtypeStruct((NPAD, out_dim), jnp.float32),
    )(a3, c3, xfeat, w3, root, bias)


def kernel(x, edge_index, edge_type, comp1, basis1, root1, bias1,
           comp2, basis2, root2, bias2):
    src = edge_index[0].astype(jnp.int32)
    dst = edge_index[1].astype(jnp.int32)
    et = edge_type.astype(jnp.int32)

    xp = jnp.pad(x, ((0, NPAD - N), (0, 0)))
    zfeat = jnp.zeros((CHR + 8, 128), jnp.float32)
    zcnt = jnp.zeros((CHR + 8, 16), jnp.float32)
    ones_h = jnp.ones((KB, 16), jnp.float32)

    h1 = root1.shape[1]
    zdim = root2.shape[1]

    comp1p = jnp.pad(comp1, ((0, 0), (0, 32 - NB_BASES)))
    comp2p = jnp.pad(comp2, ((0, 0), (0, 32 - NB_BASES)))
    b1f = jnp.pad(basis1.reshape(NB_BASES, 128 * h1), ((0, 2), (0, 0)))
    b2f = jnp.pad(basis2.reshape(NB_BASES, 128 * zdim), ((0, 2), (0, 0)))
    w1 = _wcomb(comp1p, b1f).reshape(R, 128, h1)
    w2 = _wcomb(comp2p, b2f).reshape(R, 128, zdim)

    a1, c1 = _sc_agg(x, src, dst, et, zfeat, zcnt, ones_h)
    z1 = _conv_matmul(a1.reshape(R, NPAD, 128), c1.reshape(R, NPAD, 16),
                      xp, w1, root1, bias1.reshape(1, h1), h1, leaky=True)

    a2, c2 = _sc_agg(z1, src, dst, et, zfeat, zcnt, ones_h)
    z2 = _conv_matmul(a2.reshape(R, NPAD, 128), c2.reshape(R, NPAD, 16),
                      z1, w2, root2, bias2.reshape(1, zdim), zdim, leaky=False)

    return z2[:N]


# speedup vs baseline: 5.0781x; 5.0781x over previous
import functools

import jax
import jax.numpy as jnp
from jax import lax
from jax.experimental import pallas as pl
from jax.experimental.pallas import tpu as pltpu
from jax.experimental.pallas import tpu_sc as plsc

N = 10000
E = 320000
R = 8
NB_BASES = 30

NSC = 2
NT = 16
L = 16

CH = 1008
NCHUNK = 10
CPS = NCHUNK // NSC
NPAD = NCHUNK * CH
CHR = CH * R
TRASH = CHR
ROWS_PT = CHR // NT

EPT = E // NT
SEG = 2000
NSEG = EPT // SEG
KB = 128
CAP = 17
CNTW = CHR + 16


def _sc_agg(feat, src, dst, et, zfeat):
    mesh = plsc.VectorSubcoreMesh(
        core_axis_name="c", subcore_axis_name="s",
        num_cores=NSC, num_subcores=NT)

    @functools.partial(
        pl.kernel,
        out_type=[
            jax.ShapeDtypeStruct((R * NPAD, 128), jnp.float32),
            jax.ShapeDtypeStruct((NT * R * NPAD,), jnp.float32),
        ],
        mesh=mesh,
        scratch_types=[
            pltpu.VMEM((SEG,), jnp.int32),
            pltpu.VMEM((SEG,), jnp.int32),
            pltpu.VMEM((SEG,), jnp.int32),
            pltpu.VMEM((CAP, KB), jnp.int32),
            pltpu.VMEM((CAP, KB), jnp.int32),
            pltpu.VMEM((KB, 128), jnp.float32),
            pltpu.VMEM((CNTW,), jnp.float32),
            pltpu.SMEM((8,), jnp.int32),
            pltpu.VMEM_SHARED((CHR + 8, 128), jnp.float32),
            pltpu.SemaphoreType.DMA,
        ],
        compiler_params=pltpu.CompilerParams(needs_layout_passes=False),
    )
    def k(feat_h, src_h, dst_h, et_h, zf_h, a_out, c_out,
          sbuf, dbuf, tbuf, gidx, ridx, rows, cntbuf, ssc, acc_sh, sem):
        cid = lax.axis_index("c")
        tid = lax.axis_index("s")
        iota = lax.iota(jnp.int32, L)
        zero16i = jnp.zeros((L,), jnp.int32)
        zero16f = jnp.zeros((L,), jnp.float32)
        ones16f = jnp.ones((L,), jnp.float32)
        trash16 = jnp.full((L,), TRASH, jnp.int32)
        r0 = tid * ROWS_PT

        def fire(b):
            pltpu.async_copy(feat_h.at[gidx.at[b]], rows, sem).wait()
            pltpu.sync_copy(rows, acc_sh.at[ridx.at[b]], add=True)

        def chunk_body(ci, carry):
            chunk = cid * CPS + ci
            lo = chunk * CH

            pltpu.sync_copy(zf_h.at[pl.ds(r0, ROWS_PT)],
                            acc_sh.at[pl.ds(r0, ROWS_PT)])

            @pl.when(tid == 0)
            def _():
                pltpu.sync_copy(zf_h.at[pl.ds(TRASH, 8)],
                                acc_sh.at[pl.ds(TRASH, 8)])

            def zcnt(i, c):
                cntbuf[pl.ds(i * L, L)] = zero16f
                return c

            lax.fori_loop(0, CNTW // L, zcnt, jnp.int32(0))
            plsc.subcore_barrier()

            def subround(sg, off):
                sb = tid * EPT + sg * SEG
                pltpu.sync_copy(src_h.at[pl.ds(sb, SEG)], sbuf)
                pltpu.sync_copy(dst_h.at[pl.ds(sb, SEG)], dbuf)
                pltpu.sync_copy(et_h.at[pl.ds(sb, SEG)], tbuf)

                def grp(g, off):
                    s = sbuf[pl.ds(g * L, L)]
                    d = dbuf[pl.ds(g * L, L)]
                    t = tbuf[pl.ds(g * L, L)]
                    m = (d >= lo) & (d < lo + CH)
                    rl = jnp.where(m, t * CH + (d - lo), TRASH)
                    plsc.addupdate_scatter(cntbuf, [rl], ones16f, mask=m)
                    mi = m.astype(jnp.int32)
                    idx = jnp.where(m, off + plsc.cumsum(mi) - 1, 0)
                    plsc.store_scatter(gidx, [idx >> 7, idx & 127], s, mask=m)
                    plsc.store_scatter(ridx, [idx >> 7, idx & 127], rl,
                                       mask=m)
                    return off + jnp.sum(mi)

                off = lax.fori_loop(0, SEG // L, grp, off)

                ssc[0] = off >> 7
                nfull = ssc[0]
                lax.fori_loop(0, nfull, lambda b, c: (fire(b), c)[1],
                              jnp.int32(0))

                base = nfull << 7
                for kk in range(KB // L):
                    pos = base + kk * L + iota
                    pm = pos < off
                    gv = plsc.load_gather(gidx, [pos >> 7, pos & 127],
                                          mask=pm)
                    rv = plsc.load_gather(ridx, [pos >> 7, pos & 127],
                                          mask=pm)
                    dmin = kk * L + iota
                    plsc.store_scatter(gidx, [dmin >> 7, dmin & 127], gv,
                                       mask=pm)
                    plsc.store_scatter(ridx, [dmin >> 7, dmin & 127], rv,
                                       mask=pm)
                return off - base

            off = lax.fori_loop(0, NSEG, subround, jnp.int32(0))

            pend = ((off + KB - 1) >> 7) << 7
            for kk in range(KB // L):
                pos = off + kk * L + iota
                pm = pos < pend
                plsc.store_scatter(gidx, [pos >> 7, pos & 127], zero16i,
                                   mask=pm)
                plsc.store_scatter(ridx, [pos >> 7, pos & 127], trash16,
                                   mask=pm)

            ssc[1] = off
            offs = ssc[1]

            @pl.when(offs > 0)
            def _():
                fire(jnp.int32(0))

            plsc.subcore_barrier()

            rel = tid // 2
            d0 = rel * NPAD + chunk * CH + (tid % 2) * ROWS_PT
            pltpu.sync_copy(acc_sh.at[pl.ds(r0, ROWS_PT)],
                            a_out.at[pl.ds(d0, ROWS_PT)])
            for r in range(R):
                pltpu.sync_copy(
                    cntbuf.at[pl.ds(r * CH, CH)],
                    c_out.at[pl.ds(tid * (R * NPAD) + r * NPAD + chunk * CH,
                                   CH)])
            return carry

        lax.fori_loop(0, CPS, chunk_body, jnp.int32(0))

    return k(feat, src, dst, et, zfeat)


def _wcomb(comp_p, basis_flat):
    cols = basis_flat.shape[1]

    def body(c_ref, b_ref, o_ref):
        o_ref[...] = jnp.dot(c_ref[...], b_ref[...],
                             preferred_element_type=jnp.float32)

    return pl.pallas_call(
        body,
        out_shape=jax.ShapeDtypeStruct((R, cols), jnp.float32),
    )(comp_p, basis_flat)


BLK = 1008


def _conv_matmul(a3, c4, xfeat, w3, root, bias, out_dim, leaky):
    nb = NPAD // BLK

    def body(a_ref, c_ref, x_ref, w_ref, rt_ref, b_ref, o_ref):
        r = pl.program_id(1)

        @pl.when(r == 0)
        def _():
            o_ref[...] = (
                jnp.dot(x_ref[...], rt_ref[...],
                        preferred_element_type=jnp.float32) + b_ref[...])

        a = a_ref[0]
        c = jnp.sum(c_ref[:, 0, :, :], axis=0)
        inv = 1.0 / jnp.maximum(c, 1.0)
        o_ref[...] += jnp.dot(a * inv, w_ref[0],
                              preferred_element_type=jnp.float32)

        if leaky:
            @pl.when(r == R - 1)
            def _():
                v = o_ref[...]
                o_ref[...] = jnp.where(v > 0, v, 0.01 * v)

    return pl.pallas_call(
        body,
        grid=(nb, R),
        in_specs=[
            pl.BlockSpec((1, BLK, 128), lambda i, r: (r, i, 0)),
            pl.BlockSpec((NT, 1, BLK, 1), lambda i, r: (0, r, i, 0)),
            pl.BlockSpec((BLK, 128), lambda i, r: (i, 0)),
            pl.BlockSpec((1, 128, out_dim), lambda i, r: (r, 0, 0)),
            pl.BlockSpec((128, out_dim), lambda i, r: (0, 0)),
            pl.BlockSpec((1, out_dim), lambda i, r: (0, 0)),
        ],
        out_specs=pl.BlockSpec((BLK, out_dim), lambda i, r: (i, 0)),
        out_shape=jax.ShapeDtypeStruct((NPAD, out_dim), jnp.float32),
    )(a3, c4, xfeat, w3, root, bias)


def kernel(x, edge_index, edge_type, comp1, basis1, root1, bias1,
           comp2, basis2, root2, bias2):
    src = edge_index[0].astype(jnp.int32)
    dst = edge_index[1].astype(jnp.int32)
    et = edge_type.astype(jnp.int32)

    xp = jnp.pad(x, ((0, NPAD - N), (0, 0)))
    zfeat = jnp.zeros((CHR + 8, 128), jnp.float32)

    h1 = root1.shape[1]
    zdim = root2.shape[1]

    comp1p = jnp.pad(comp1, ((0, 0), (0, 32 - NB_BASES)))
    comp2p = jnp.pad(comp2, ((0, 0), (0, 32 - NB_BASES)))
    b1f = jnp.pad(basis1.reshape(NB_BASES, 128 * h1), ((0, 2), (0, 0)))
    b2f = jnp.pad(basis2.reshape(NB_BASES, 128 * zdim), ((0, 2), (0, 0)))
    w1 = _wcomb(comp1p, b1f).reshape(R, 128, h1)
    w2 = _wcomb(comp2p, b2f).reshape(R, 128, zdim)

    a1, c1 = _sc_agg(x, src, dst, et, zfeat)
    z1 = _conv_matmul(a1.reshape(R, NPAD, 128),
                      c1.reshape(NT, R, NPAD, 1),
                      xp, w1, root1, bias1.reshape(1, h1), h1, leaky=True)

    a2, c2 = _sc_agg(z1, src, dst, et, zfeat)
    z2 = _conv_matmul(a2.reshape(R, NPAD, 128),
                      c2.reshape(NT, R, NPAD, 1),
                      z1, w2, root2, bias2.reshape(1, zdim), zdim,
                      leaky=False)

    return z2[:N]


# speedup vs baseline: 5.3697x; 1.0574x over previous
import functools

import jax
import jax.numpy as jnp
from jax import lax
from jax.experimental import pallas as pl
from jax.experimental.pallas import tpu as pltpu
from jax.experimental.pallas import tpu_sc as plsc

N = 10000
E = 320000
R = 8
NB_BASES = 30

NSC = 2
NT = 16
L = 16

CH = 1008
NCHUNK = 10
CPS = NCHUNK // NSC
NPAD = NCHUNK * CH
CHR = CH * R
TRASH = CHR
ROWS_PT = CHR // NT

EPT = E // NT
SEG = 2000
NSEG = EPT // SEG
KB = 128
CAP = 17
CNTW = CHR + 16


def _sc_agg(feat, src, dst, et, zfeat):
    mesh = plsc.VectorSubcoreMesh(
        core_axis_name="c", subcore_axis_name="s",
        num_cores=NSC, num_subcores=NT)

    @functools.partial(
        pl.kernel,
        out_type=[
            jax.ShapeDtypeStruct((R * NPAD, 128), jnp.float32),
            jax.ShapeDtypeStruct((NT * R * NPAD,), jnp.float32),
        ],
        mesh=mesh,
        scratch_types=[
            pltpu.VMEM((SEG,), jnp.int32),
            pltpu.VMEM((SEG,), jnp.int32),
            pltpu.VMEM((SEG,), jnp.int32),
            pltpu.VMEM((SEG,), jnp.int32),
            pltpu.VMEM((SEG,), jnp.int32),
            pltpu.VMEM((SEG,), jnp.int32),
            pltpu.VMEM((CAP, KB), jnp.int32),
            pltpu.VMEM((CAP, KB), jnp.int32),
            pltpu.VMEM((KB, 128), jnp.float32),
            pltpu.VMEM((CNTW,), jnp.float32),
            pltpu.SMEM((8,), jnp.int32),
            pltpu.VMEM_SHARED((CHR + 8, 128), jnp.float32),
            pltpu.SemaphoreType.DMA,
            pltpu.SemaphoreType.DMA,
            pltpu.SemaphoreType.DMA,
            pltpu.SemaphoreType.DMA,
        ],
        compiler_params=pltpu.CompilerParams(needs_layout_passes=False),
    )
    def k(feat_h, src_h, dst_h, et_h, zf_h, a_out, c_out,
          sbuf0, dbuf0, tbuf0, sbuf1, dbuf1, tbuf1,
          gidx, ridx, rows, cntbuf, ssc, acc_sh, sem, esem_s, esem_d, esem_t):
        cid = lax.axis_index("c")
        tid = lax.axis_index("s")
        iota = lax.iota(jnp.int32, L)
        zero16i = jnp.zeros((L,), jnp.int32)
        zero16f = jnp.zeros((L,), jnp.float32)
        ones16f = jnp.ones((L,), jnp.float32)
        trash16 = jnp.full((L,), TRASH, jnp.int32)
        r0 = tid * ROWS_PT

        def fire(b):
            pltpu.async_copy(feat_h.at[gidx.at[b]], rows, sem).wait()
            pltpu.sync_copy(rows, acc_sh.at[ridx.at[b]], add=True)

        B0 = (sbuf0, dbuf0, tbuf0)
        B1 = (sbuf1, dbuf1, tbuf1)

        def eissue(bufs, sg):
            sb = tid * EPT + sg * SEG
            pltpu.async_copy(src_h.at[pl.ds(sb, SEG)], bufs[0], esem_s)
            pltpu.async_copy(dst_h.at[pl.ds(sb, SEG)], bufs[1], esem_d)
            pltpu.async_copy(et_h.at[pl.ds(sb, SEG)], bufs[2], esem_t)

        def ewait(bufs, sg):
            sb = tid * EPT + sg * SEG
            pltpu.make_async_copy(src_h.at[pl.ds(sb, SEG)], bufs[0],
                                  esem_s).wait()
            pltpu.make_async_copy(dst_h.at[pl.ds(sb, SEG)], bufs[1],
                                  esem_d).wait()
            pltpu.make_async_copy(et_h.at[pl.ds(sb, SEG)], bufs[2],
                                  esem_t).wait()

        def chunk_body(ci, carry):
            chunk = cid * CPS + ci
            lo = chunk * CH

            pltpu.sync_copy(zf_h.at[pl.ds(r0, ROWS_PT)],
                            acc_sh.at[pl.ds(r0, ROWS_PT)])

            @pl.when(tid == 0)
            def _():
                pltpu.sync_copy(zf_h.at[pl.ds(TRASH, 8)],
                                acc_sh.at[pl.ds(TRASH, 8)])

            def zcnt(i, c):
                cntbuf[pl.ds(i * L, L)] = zero16f
                return c

            lax.fori_loop(0, CNTW // L, zcnt, jnp.int32(0))
            plsc.subcore_barrier()

            def halfround(bufs, off):
                sb_, db_, tb_ = bufs

                def grp(g, off):
                    s = sb_[pl.ds(g * L, L)]
                    d = db_[pl.ds(g * L, L)]
                    t = tb_[pl.ds(g * L, L)]
                    m = (d >= lo) & (d < lo + CH)
                    rl = jnp.where(m, t * CH + (d - lo), TRASH)
                    plsc.addupdate_scatter(cntbuf, [rl], ones16f, mask=m)
                    mi = m.astype(jnp.int32)
                    idx = jnp.where(m, off + plsc.cumsum(mi) - 1, 0)
                    plsc.store_scatter(gidx, [idx >> 7, idx & 127], s, mask=m)
                    plsc.store_scatter(ridx, [idx >> 7, idx & 127], rl,
                                       mask=m)
                    return off + jnp.sum(mi)

                off = lax.fori_loop(0, SEG // L, grp, off)

                ssc[0] = off >> 7
                nfull = ssc[0]
                lax.fori_loop(0, nfull, lambda b, c: (fire(b), c)[1],
                              jnp.int32(0))

                base = nfull << 7
                for kk in range(KB // L):
                    pos = base + kk * L + iota
                    pm = pos < off
                    gv = plsc.load_gather(gidx, [pos >> 7, pos & 127],
                                          mask=pm)
                    rv = plsc.load_gather(ridx, [pos >> 7, pos & 127],
                                          mask=pm)
                    dmin = kk * L + iota
                    plsc.store_scatter(gidx, [dmin >> 7, dmin & 127], gv,
                                       mask=pm)
                    plsc.store_scatter(ridx, [dmin >> 7, dmin & 127], rv,
                                       mask=pm)
                return off - base

            def pair(p, off):
                sg0 = 2 * p
                ewait(B0, sg0)
                eissue(B1, sg0 + 1)
                off = halfround(B0, off)
                ewait(B1, sg0 + 1)

                @pl.when(jnp.logical_not((ci == CPS - 1)
                                         & (p == NSEG // 2 - 1)))
                def _():
                    eissue(B0, (sg0 + 2) % NSEG)

                return halfround(B1, off)

            off = lax.fori_loop(0, NSEG // 2, pair, jnp.int32(0))

            pend = ((off + KB - 1) >> 7) << 7
            for kk in range(KB // L):
                pos = off + kk * L + iota
                pm = pos < pend
                plsc.store_scatter(gidx, [pos >> 7, pos & 127], zero16i,
                                   mask=pm)
                plsc.store_scatter(ridx, [pos >> 7, pos & 127], trash16,
                                   mask=pm)

            ssc[1] = off
            offs = ssc[1]

            @pl.when(offs > 0)
            def _():
                fire(jnp.int32(0))

            plsc.subcore_barrier()

            rel = tid // 2
            d0 = rel * NPAD + chunk * CH + (tid % 2) * ROWS_PT
            pltpu.sync_copy(acc_sh.at[pl.ds(r0, ROWS_PT)],
                            a_out.at[pl.ds(d0, ROWS_PT)])
            for r in range(R):
                pltpu.sync_copy(
                    cntbuf.at[pl.ds(r * CH, CH)],
                    c_out.at[pl.ds(tid * (R * NPAD) + r * NPAD + chunk * CH,
                                   CH)])
            return carry

        eissue(B0, 0)
        lax.fori_loop(0, CPS, chunk_body, jnp.int32(0))

    return k(feat, src, dst, et, zfeat)


def _wcomb(comp_p, basis_flat):
    cols = basis_flat.shape[1]

    def body(c_ref, b_ref, o_ref):
        o_ref[...] = jnp.dot(c_ref[...], b_ref[...],
                             preferred_element_type=jnp.float32)

    return pl.pallas_call(
        body,
        out_shape=jax.ShapeDtypeStruct((R, cols), jnp.float32),
    )(comp_p, basis_flat)


BLK = 1008


def _conv_matmul(a3, c4, xfeat, w3, root, bias, out_dim, leaky):
    nb = NPAD // BLK

    def body(a_ref, c_ref, x_ref, w_ref, rt_ref, b_ref, o_ref):
        r = pl.program_id(1)

        @pl.when(r == 0)
        def _():
            o_ref[...] = (
                jnp.dot(x_ref[...], rt_ref[...],
                        preferred_element_type=jnp.float32) + b_ref[...])

        a = a_ref[0]
        c = jnp.sum(c_ref[:, 0, :, :], axis=0)
        inv = 1.0 / jnp.maximum(c, 1.0)
        o_ref[...] += jnp.dot(a * inv, w_ref[0],
                              preferred_element_type=jnp.float32)

        if leaky:
            @pl.when(r == R - 1)
            def _():
                v = o_ref[...]
                o_ref[...] = jnp.where(v > 0, v, 0.01 * v)

    return pl.pallas_call(
        body,
        grid=(nb, R),
        in_specs=[
            pl.BlockSpec((1, BLK, 128), lambda i, r: (r, i, 0)),
            pl.BlockSpec((NT, 1, BLK, 1), lambda i, r: (0, r, i, 0)),
            pl.BlockSpec((BLK, 128), lambda i, r: (i, 0)),
            pl.BlockSpec((1, 128, out_dim), lambda i, r: (r, 0, 0)),
            pl.BlockSpec((128, out_dim), lambda i, r: (0, 0)),
            pl.BlockSpec((1, out_dim), lambda i, r: (0, 0)),
        ],
        out_specs=pl.BlockSpec((BLK, out_dim), lambda i, r: (i, 0)),
        out_shape=jax.ShapeDtypeStruct((NPAD, out_dim), jnp.float32),
    )(a3, c4, xfeat, w3, root, bias)


def kernel(x, edge_index, edge_type, comp1, basis1, root1, bias1,
           comp2, basis2, root2, bias2):
    src = edge_index[0].astype(jnp.int32)
    dst = edge_index[1].astype(jnp.int32)
    et = edge_type.astype(jnp.int32)

    xp = jnp.pad(x, ((0, NPAD - N), (0, 0)))
    zfeat = jnp.zeros((CHR + 8, 128), jnp.float32)

    h1 = root1.shape[1]
    zdim = root2.shape[1]

    comp1p = jnp.pad(comp1, ((0, 0), (0, 32 - NB_BASES)))
    comp2p = jnp.pad(comp2, ((0, 0), (0, 32 - NB_BASES)))
    b1f = jnp.pad(basis1.reshape(NB_BASES, 128 * h1), ((0, 2), (0, 0)))
    b2f = jnp.pad(basis2.reshape(NB_BASES, 128 * zdim), ((0, 2), (0, 0)))
    w1 = _wcomb(comp1p, b1f).reshape(R, 128, h1)
    w2 = _wcomb(comp2p, b2f).reshape(R, 128, zdim)

    a1, c1 = _sc_agg(x, src, dst, et, zfeat)
    z1 = _conv_matmul(a1.reshape(R, NPAD, 128),
                      c1.reshape(NT, R, NPAD, 1),
                      xp, w1, root1, bias1.reshape(1, h1), h1, leaky=True)

    a2, c2 = _sc_agg(z1, src, dst, et, zfeat)
    z2 = _conv_matmul(a2.reshape(R, NPAD, 128),
                      c2.reshape(NT, R, NPAD, 1),
                      z1, w2, root2, bias2.reshape(1, zdim), zdim,
                      leaky=False)

    return z2[:N]


# speedup vs baseline: 6.8240x; 1.2708x over previous
import functools

import jax
import jax.numpy as jnp
from jax import lax
from jax.experimental import pallas as pl
from jax.experimental.pallas import tpu as pltpu
from jax.experimental.pallas import tpu_sc as plsc

N = 10000
E = 320000
R = 8
NB_BASES = 30

NSC = 2
NT = 16
L = 16

CH = 1008
NCHUNK = 10
CPS = NCHUNK // NSC
NPAD = NCHUNK * CH
CHR = CH * R
TRASH = CHR
ROWS_PT = CHR // NT

EPT = E // NT
SEG = 2000
NSEG = EPT // SEG
KB = 128
CAP = 17
CNTW = CHR + 16


def _sc_agg(feat, src, dst, et, zfeat, with_counts):
    mesh = plsc.VectorSubcoreMesh(
        core_axis_name="c", subcore_axis_name="s",
        num_cores=NSC, num_subcores=NT)

    out_type = [jax.ShapeDtypeStruct((R * NPAD, 128), jnp.float32)]
    if with_counts:
        out_type.append(jax.ShapeDtypeStruct((NT * R * NPAD,), jnp.float32))

    @functools.partial(
        pl.kernel,
        out_type=out_type,
        mesh=mesh,
        scratch_types=[
            pltpu.VMEM((SEG,), jnp.int32),
            pltpu.VMEM((SEG,), jnp.int32),
            pltpu.VMEM((SEG,), jnp.int32),
            pltpu.VMEM((SEG,), jnp.int32),
            pltpu.VMEM((SEG,), jnp.int32),
            pltpu.VMEM((SEG,), jnp.int32),
            pltpu.VMEM((CAP, KB), jnp.int32),
            pltpu.VMEM((CAP, KB), jnp.int32),
            pltpu.VMEM((KB, 128), jnp.float32),
            pltpu.VMEM((CNTW,), jnp.float32),
            pltpu.SMEM((8,), jnp.int32),
            pltpu.VMEM_SHARED((CHR + 8, 128), jnp.float32),
            pltpu.SemaphoreType.DMA,
            pltpu.SemaphoreType.DMA,
            pltpu.SemaphoreType.DMA,
            pltpu.SemaphoreType.DMA,
        ],
        compiler_params=pltpu.CompilerParams(needs_layout_passes=False),
    )
    def k(*refs):
        if with_counts:
            (feat_h, src_h, dst_h, et_h, zf_h, a_out, c_out,
             sbuf0, dbuf0, tbuf0, sbuf1, dbuf1, tbuf1,
             gidx, ridx, rows, cntbuf, ssc, acc_sh,
             sem, esem_s, esem_d, esem_t) = refs
        else:
            (feat_h, src_h, dst_h, et_h, zf_h, a_out,
             sbuf0, dbuf0, tbuf0, sbuf1, dbuf1, tbuf1,
             gidx, ridx, rows, cntbuf, ssc, acc_sh,
             sem, esem_s, esem_d, esem_t) = refs
            c_out = None
        cid = lax.axis_index("c")
        tid = lax.axis_index("s")
        iota = lax.iota(jnp.int32, L)
        zero16i = jnp.zeros((L,), jnp.int32)
        zero16f = jnp.zeros((L,), jnp.float32)
        ones16f = jnp.ones((L,), jnp.float32)
        trash16 = jnp.full((L,), TRASH, jnp.int32)
        r0 = tid * ROWS_PT

        def fire(b):
            pltpu.async_copy(feat_h.at[gidx.at[b]], rows, sem).wait()
            pltpu.sync_copy(rows, acc_sh.at[ridx.at[b]], add=True)

        B0 = (sbuf0, dbuf0, tbuf0)
        B1 = (sbuf1, dbuf1, tbuf1)

        def eissue(bufs, sg):
            sb = tid * EPT + sg * SEG
            pltpu.async_copy(src_h.at[pl.ds(sb, SEG)], bufs[0], esem_s)
            pltpu.async_copy(dst_h.at[pl.ds(sb, SEG)], bufs[1], esem_d)
            pltpu.async_copy(et_h.at[pl.ds(sb, SEG)], bufs[2], esem_t)

        def ewait(bufs, sg):
            sb = tid * EPT + sg * SEG
            pltpu.make_async_copy(src_h.at[pl.ds(sb, SEG)], bufs[0],
                                  esem_s).wait()
            pltpu.make_async_copy(dst_h.at[pl.ds(sb, SEG)], bufs[1],
                                  esem_d).wait()
            pltpu.make_async_copy(et_h.at[pl.ds(sb, SEG)], bufs[2],
                                  esem_t).wait()

        def chunk_body(ci, carry):
            chunk = cid * CPS + ci
            lo = chunk * CH

            pltpu.sync_copy(zf_h.at[pl.ds(r0, ROWS_PT)],
                            acc_sh.at[pl.ds(r0, ROWS_PT)])

            @pl.when(tid == 0)
            def _():
                pltpu.sync_copy(zf_h.at[pl.ds(TRASH, 8)],
                                acc_sh.at[pl.ds(TRASH, 8)])

            if with_counts:
                def zcnt(i, c):
                    cntbuf[pl.ds(i * L, L)] = zero16f
                    return c

                lax.fori_loop(0, CNTW // L, zcnt, jnp.int32(0))
            plsc.subcore_barrier()

            def halfround(bufs, off):
                sb_, db_, tb_ = bufs

                def grp(g, off):
                    s = sb_[pl.ds(g * L, L)]
                    d = db_[pl.ds(g * L, L)]
                    t = tb_[pl.ds(g * L, L)]
                    m = (d >= lo) & (d < lo + CH)
                    rl = jnp.where(m, t * CH + (d - lo), TRASH)
                    if with_counts:
                        plsc.addupdate_scatter(cntbuf, [rl], ones16f, mask=m)
                    mi = m.astype(jnp.int32)
                    idx = jnp.where(m, off + plsc.cumsum(mi) - 1, 0)
                    plsc.store_scatter(gidx, [idx >> 7, idx & 127], s, mask=m)
                    plsc.store_scatter(ridx, [idx >> 7, idx & 127], rl,
                                       mask=m)
                    return off + jnp.sum(mi)

                off = lax.fori_loop(0, SEG // L, grp, off)

                ssc[0] = off >> 7
                nfull = ssc[0]
                lax.fori_loop(0, nfull, lambda b, c: (fire(b), c)[1],
                              jnp.int32(0))

                base = nfull << 7
                for kk in range(KB // L):
                    pos = base + kk * L + iota
                    pm = pos < off
                    gv = plsc.load_gather(gidx, [pos >> 7, pos & 127],
                                          mask=pm)
                    rv = plsc.load_gather(ridx, [pos >> 7, pos & 127],
                                          mask=pm)
                    dmin = kk * L + iota
                    plsc.store_scatter(gidx, [dmin >> 7, dmin & 127], gv,
                                       mask=pm)
                    plsc.store_scatter(ridx, [dmin >> 7, dmin & 127], rv,
                                       mask=pm)
                return off - base

            def pair(p, off):
                sg0 = 2 * p
                ewait(B0, sg0)
                eissue(B1, sg0 + 1)
                off = halfround(B0, off)
                ewait(B1, sg0 + 1)

                @pl.when(jnp.logical_not((ci == CPS - 1)
                                         & (p == NSEG // 2 - 1)))
                def _():
                    eissue(B0, (sg0 + 2) % NSEG)

                return halfround(B1, off)

            off = lax.fori_loop(0, NSEG // 2, pair, jnp.int32(0))

            pend = ((off + KB - 1) >> 7) << 7
            for kk in range(KB // L):
                pos = off + kk * L + iota
                pm = pos < pend
                plsc.store_scatter(gidx, [pos >> 7, pos & 127], zero16i,
                                   mask=pm)
                plsc.store_scatter(ridx, [pos >> 7, pos & 127], trash16,
                                   mask=pm)

            ssc[1] = off
            offs = ssc[1]

            @pl.when(offs > 0)
            def _():
                fire(jnp.int32(0))

            plsc.subcore_barrier()

            rel = tid // 2
            d0 = rel * NPAD + chunk * CH + (tid % 2) * ROWS_PT
            pltpu.sync_copy(acc_sh.at[pl.ds(r0, ROWS_PT)],
                            a_out.at[pl.ds(d0, ROWS_PT)])
            if with_counts:
                for r in range(R):
                    pltpu.sync_copy(
                        cntbuf.at[pl.ds(r * CH, CH)],
                        c_out.at[pl.ds(tid * (R * NPAD) + r * NPAD
                                       + chunk * CH, CH)])
            return carry

        eissue(B0, 0)
        lax.fori_loop(0, CPS, chunk_body, jnp.int32(0))

    return k(feat, src, dst, et, zfeat)


def _wcomb(comp_p, basis_flat):
    cols = basis_flat.shape[1]

    def body(c_ref, b_ref, o_ref):
        o_ref[...] = jnp.dot(c_ref[...], b_ref[...],
                             preferred_element_type=jnp.float32)

    return pl.pallas_call(
        body,
        out_shape=jax.ShapeDtypeStruct((R, cols), jnp.float32),
    )(comp_p, basis_flat)


BLK = 1008


def _conv_matmul(a3, c4, xfeat, w3, root, bias, out_dim, leaky):
    nb = NPAD // BLK

    def body(a_ref, c_ref, x_ref, w_ref, rt_ref, b_ref, o_ref):
        r = pl.program_id(1)

        @pl.when(r == 0)
        def _():
            o_ref[...] = (
                jnp.dot(x_ref[...], rt_ref[...],
                        preferred_element_type=jnp.float32) + b_ref[...])

        a = a_ref[0]
        c = jnp.sum(c_ref[:, 0, :, :], axis=0)
        inv = 1.0 / jnp.maximum(c, 1.0)
        o_ref[...] += jnp.dot(a * inv, w_ref[0],
                              preferred_element_type=jnp.float32)

        if leaky:
            @pl.when(r == R - 1)
            def _():
                v = o_ref[...]
                o_ref[...] = jnp.where(v > 0, v, 0.01 * v)

    return pl.pallas_call(
        body,
        grid=(nb, R),
        in_specs=[
            pl.BlockSpec((1, BLK, 128), lambda i, r: (r, i, 0)),
            pl.BlockSpec((NT, 1, BLK, 1), lambda i, r: (0, r, i, 0)),
            pl.BlockSpec((BLK, 128), lambda i, r: (i, 0)),
            pl.BlockSpec((1, 128, out_dim), lambda i, r: (r, 0, 0)),
            pl.BlockSpec((128, out_dim), lambda i, r: (0, 0)),
            pl.BlockSpec((1, out_dim), lambda i, r: (0, 0)),
        ],
        out_specs=pl.BlockSpec((BLK, out_dim), lambda i, r: (i, 0)),
        out_shape=jax.ShapeDtypeStruct((NPAD, out_dim), jnp.float32),
    )(a3, c4, xfeat, w3, root, bias)


def kernel(x, edge_index, edge_type, comp1, basis1, root1, bias1,
           comp2, basis2, root2, bias2):
    src = edge_index[0].astype(jnp.int32)
    dst = edge_index[1].astype(jnp.int32)
    et = edge_type.astype(jnp.int32)

    xp = jnp.pad(x, ((0, NPAD - N), (0, 0)))
    zfeat = jnp.zeros((CHR + 8, 128), jnp.float32)

    h1 = root1.shape[1]
    zdim = root2.shape[1]

    comp1p = jnp.pad(comp1, ((0, 0), (0, 32 - NB_BASES)))
    comp2p = jnp.pad(comp2, ((0, 0), (0, 32 - NB_BASES)))
    b1f = jnp.pad(basis1.reshape(NB_BASES, 128 * h1), ((0, 2), (0, 0)))
    b2f = jnp.pad(basis2.reshape(NB_BASES, 128 * zdim), ((0, 2), (0, 0)))
    w1 = _wcomb(comp1p, b1f).reshape(R, 128, h1)
    w2 = _wcomb(comp2p, b2f).reshape(R, 128, zdim)

    a1, c1 = _sc_agg(x, src, dst, et, zfeat, with_counts=True)
    c4 = c1.reshape(NT, R, NPAD, 1)
    z1 = _conv_matmul(a1.reshape(R, NPAD, 128), c4,
                      xp, w1, root1, bias1.reshape(1, h1), h1, leaky=True)

    (a2,) = _sc_agg(z1, src, dst, et, zfeat, with_counts=False)
    z2 = _conv_matmul(a2.reshape(R, NPAD, 128), c4,
                      z1, w2, root2, bias2.reshape(1, zdim), zdim,
                      leaky=False)

    return z2[:N]
